# Initial kernel scaffold; baseline (speedup 1.0000x reference)
import jax
import jax.numpy as jnp
from jax.experimental import pallas as pl


def kernel(user_table, item_table, edge_index, user_id, topk):
    raise NotImplementedError("write your pallas kernel here")


# speedup vs baseline: 7.4636x; 7.4636x over previous
import jax
import jax.numpy as jnp
from jax import lax
from jax.experimental import pallas as pl
from jax.experimental.pallas import tpu as pltpu
from jax.experimental.pallas import tpu_sc as plsc

NUM_USERS = 10000
NUM_ITEMS = 40000
F = 64
FH = 32
N = NUM_USERS + NUM_ITEMS
E = 800000
NC = 2
NS = 16
NP = 50048
ROWS_PER_TILE = NP // NS
EP = 802816
ER = EP // 128
ER_PER_TILE = ER // NS
CH = 4
NCHUNK = ER_PER_TILE // CH
DW = 8
ER_PER_CORE = ER // NC
ERD_PER_TILE = ER_PER_CORE // NS
DCH = 4
NDCHUNK = ERD_PER_TILE // DCH
TOPK = 10
NEG = -1e30

_SC_PARAMS = pltpu.CompilerParams(use_tc_tiling_on_sc=False)


def _make_mesh():
    return plsc.VectorSubcoreMesh(
        core_axis_name="c", subcore_axis_name="s", num_cores=NC
    )


def _make_deg():
    def body(dstm, zrows, ones_h, out0, out1, degacc, dst_v, ones_v, sem):
        c = lax.axis_index("c")
        s = lax.axis_index("s")
        r0 = s * ROWS_PER_TILE
        pltpu.sync_copy(zrows, degacc.at[pl.ds(r0, ROWS_PER_TILE)])
        pltpu.sync_copy(ones_h, ones_v)
        plsc.subcore_barrier()

        def chunk(g, carry):
            rr = c * ER_PER_CORE + s * ERD_PER_TILE + g * DCH
            pltpu.sync_copy(dstm.at[pl.ds(rr, DCH)], dst_v)
            for j in range(DCH):
                pltpu.sync_copy(ones_v, degacc.at[dst_v.at[j]], add=True)
            return carry

        lax.fori_loop(0, NDCHUNK, chunk, 0)
        plsc.subcore_barrier()

        @pl.when(c == 0)
        def _():
            pltpu.sync_copy(degacc.at[pl.ds(r0, ROWS_PER_TILE)],
                            out0.at[pl.ds(r0, ROWS_PER_TILE)])

        @pl.when(c == 1)
        def _():
            pltpu.sync_copy(degacc.at[pl.ds(r0, ROWS_PER_TILE)],
                            out1.at[pl.ds(r0, ROWS_PER_TILE)])

    return pl.kernel(
        body,
        out_type=[
            jax.ShapeDtypeStruct((NP, DW), jnp.float32),
            jax.ShapeDtypeStruct((NP, DW), jnp.float32),
        ],
        mesh=_make_mesh(),
        scratch_types=[
            pltpu.VMEM_SHARED((NP, DW), jnp.float32),
            pltpu.VMEM((DCH, 128), jnp.int32),
            pltpu.VMEM((128, DW), jnp.float32),
            pltpu.SemaphoreType.DMA,
        ],
        compiler_params=_SC_PARAMS,
    )


def _make_propagate():
    def body(x0, x1, srcm, dstm, zrows,
             out0, out1,
             acc, src_v, dst_v, rows_v, sem):
        c = lax.axis_index("c")
        s = lax.axis_index("s")
        r0 = s * ROWS_PER_TILE
        pltpu.sync_copy(zrows, acc.at[pl.ds(r0, ROWS_PER_TILE)])
        plsc.subcore_barrier()

        def make_chunk(tab):
            def chunk(g, carry):
                rr = s * ER_PER_TILE + g * CH
                pltpu.sync_copy(srcm.at[pl.ds(rr, CH)], src_v)
                pltpu.sync_copy(dstm.at[pl.ds(rr, CH)], dst_v)
                cps = [
                    pltpu.async_copy(tab.at[src_v.at[j]], rows_v.at[j], sem)
                    for j in range(CH)
                ]
                for cp in cps:
                    cp.wait()
                for j in range(CH):
                    pltpu.sync_copy(rows_v.at[j], acc.at[dst_v.at[j]], add=True)
                return carry

            return chunk

        @pl.when(c == 0)
        def _():
            lax.fori_loop(0, NCHUNK, make_chunk(x0), 0)

        @pl.when(c == 1)
        def _():
            lax.fori_loop(0, NCHUNK, make_chunk(x1), 0)

        plsc.subcore_barrier()

        @pl.when(c == 0)
        def _():
            pltpu.sync_copy(acc.at[pl.ds(r0, ROWS_PER_TILE)],
                            out0.at[pl.ds(r0, ROWS_PER_TILE)])

        @pl.when(c == 1)
        def _():
            pltpu.sync_copy(acc.at[pl.ds(r0, ROWS_PER_TILE)],
                            out1.at[pl.ds(r0, ROWS_PER_TILE)])

    return pl.kernel(
        body,
        out_type=[
            jax.ShapeDtypeStruct((NP, FH), jnp.float32),
            jax.ShapeDtypeStruct((NP, FH), jnp.float32),
        ],
        mesh=_make_mesh(),
        scratch_types=[
            pltpu.VMEM_SHARED((NP, FH), jnp.float32),
            pltpu.VMEM((CH, 128), jnp.int32),
            pltpu.VMEM((CH, 128), jnp.int32),
            pltpu.VMEM((CH, 128, FH), jnp.float32),
            pltpu.SemaphoreType.DMA,
        ],
        compiler_params=_SC_PARAMS,
    )


_deg = _make_deg()
_prop = _make_propagate()


def _divide(a0, a1, deg0, deg1):
    R = NP // 8

    def body(a0_ref, a1_ref, d0_ref, d1_ref, h0_ref, h1_ref, r_ref):
        recip = 1.0 / jnp.maximum(d0_ref[...] + d1_ref[...], 1.0)
        h0_ref[...] = a0_ref[...] * recip
        h1_ref[...] = a1_ref[...] * recip
        r_ref[...] = recip

    row = lambda i: (i, 0)
    return pl.pallas_call(
        body,
        grid=(NP // R,),
        in_specs=[
            pl.BlockSpec((R, FH), row),
            pl.BlockSpec((R, FH), row),
            pl.BlockSpec((R, 1), row),
            pl.BlockSpec((R, 1), row),
        ],
        out_specs=[
            pl.BlockSpec((R, FH), row),
            pl.BlockSpec((R, FH), row),
            pl.BlockSpec((R, 1), row),
        ],
        out_shape=[
            jax.ShapeDtypeStruct((NP, FH), jnp.float32),
            jax.ShapeDtypeStruct((NP, FH), jnp.float32),
            jax.ShapeDtypeStruct((NP, 1), jnp.float32),
        ],
    )(a0, a1, deg0, deg1)


def _scores(x0i, h0i, b0i, x1i, h1i, b1i, recipi,
            xu0, hu0, bu0, xu1, hu1, bu1, recipu):
    R = 4000

    def body(x0r, h0r, b0r, x1r, h1r, b1r, rr,
             xu0r, hu0r, bu0r, xu1r, hu1r, bu1r, rur, out):
        ru = rr[...]
        u0 = (xu0r[...] + hu0r[...] + bu0r[...] * rur[...]) / 3.0
        u1 = (xu1r[...] + hu1r[...] + bu1r[...] * rur[...]) / 3.0
        f0 = (x0r[...] + h0r[...] + b0r[...] * ru) / 3.0
        f1 = (x1r[...] + h1r[...] + b1r[...] * ru) / 3.0
        out[...] = jnp.sum(f0 * u0 + f1 * u1, axis=1, keepdims=True)

    row = lambda i: (i, 0)
    rep = lambda i: (0, 0)
    return pl.pallas_call(
        body,
        grid=(NUM_ITEMS // R,),
        in_specs=[pl.BlockSpec((R, FH), row)] * 6
        + [pl.BlockSpec((R, 1), row)]
        + [pl.BlockSpec((1, FH), rep)] * 6
        + [pl.BlockSpec((1, 1), rep)],
        out_specs=pl.BlockSpec((R, 1), row),
        out_shape=jax.ShapeDtypeStruct((NUM_ITEMS, 1), jnp.float32),
    )(x0i, h0i, b0i, x1i, h1i, b1i, recipi, xu0, hu0, bu0, xu1, hu1, bu1, recipu)


def _topk(s2d):
    def body(s_ref, vals_ref, idx_ref):
        s = s_ref[...]
        ridx = lax.broadcasted_iota(jnp.int32, s.shape, 0)
        cidx = lax.broadcasted_iota(jnp.int32, s.shape, 1)
        flat = ridx * 128 + cidx
        lane = lax.broadcasted_iota(jnp.int32, (1, 128), 1)
        vals = jnp.zeros((1, 128), jnp.float32)
        idxs = jnp.zeros((1, 128), jnp.int32)
        for k in range(TOPK):
            m = jnp.max(s)
            j = jnp.min(jnp.where(s == m, flat, jnp.int32(2**31 - 1)))
            vals = jnp.where(lane == k, m, vals)
            idxs = jnp.where(lane == k, j, idxs)
            s = jnp.where(flat == j, NEG, s)
        vals_ref[...] = vals
        idx_ref[...] = idxs

    return pl.pallas_call(
        body,
        out_shape=[
            jax.ShapeDtypeStruct((1, 128), jnp.float32),
            jax.ShapeDtypeStruct((1, 128), jnp.int32),
        ],
    )(s2d)


def kernel(user_table, item_table, edge_index, user_id, topk):
    del topk
    x = jnp.concatenate([user_table, item_table], axis=0)
    x0 = jnp.pad(x[:, :FH], ((0, NP - N), (0, 0)))
    x1 = jnp.pad(x[:, FH:], ((0, NP - N), (0, 0)))
    pad_e = jnp.full((EP - E,), NP - 1, dtype=jnp.int32)
    srcm = jnp.concatenate([edge_index[0], pad_e]).reshape(ER, 128)
    dstm = jnp.concatenate([edge_index[1], pad_e]).reshape(ER, 128)
    zrows = jnp.zeros((ROWS_PER_TILE, FH), jnp.float32)
    zdeg = jnp.zeros((ROWS_PER_TILE, DW), jnp.float32)
    ones = jnp.ones((128, DW), jnp.float32)

    deg0, deg1 = _deg(dstm, zdeg, ones)
    a0, a1 = _prop(x0, x1, srcm, dstm, zrows)
    h0, h1, recip = _divide(a0, a1, deg0[:, :1], deg1[:, :1])
    b0, b1 = _prop(h0, h1, srcm, dstm, zrows)

    uid = jnp.asarray(user_id, jnp.int32)
    sl = lambda arr: lax.dynamic_slice_in_dim(arr, uid, 1)
    scores = _scores(
        x0[NUM_USERS:N], h0[NUM_USERS:N], b0[NUM_USERS:N],
        x1[NUM_USERS:N], h1[NUM_USERS:N], b1[NUM_USERS:N], recip[NUM_USERS:N],
        sl(x0), sl(h0), sl(b0), sl(x1), sl(h1), sl(b1), sl(recip),
    )
    s2d = jnp.pad(scores[:, 0], (0, 40064 - NUM_ITEMS),
                  constant_values=NEG).reshape(313, 128)
    vals, idx = _topk(s2d)
    return vals[0, :TOPK], idx[0, :TOPK]


# speedup vs baseline: 9.1928x; 1.2317x over previous
import jax
import jax.numpy as jnp
from jax import lax
from jax.experimental import pallas as pl
from jax.experimental.pallas import tpu as pltpu
from jax.experimental.pallas import tpu_sc as plsc

NUM_USERS = 10000
NUM_ITEMS = 40000
F = 64
FH = 32
N = NUM_USERS + NUM_ITEMS
E = 800000
NC = 2
NS = 16
NP = 50048
ROWS_PER_TILE = NP // NS
EP = 802816
ER = EP // 128
ER_PER_TILE = ER // NS
CH = 2
NCHUNK = ER_PER_TILE // CH
BCH = 4
NBLK = NCHUNK // BCH
BR = BCH * CH
DW = 8
ER_PER_CORE = ER // NC
ERD_PER_TILE = ER_PER_CORE // NS
DCH = 4
NDCHUNK = ERD_PER_TILE // DCH
TOPK = 10
NEG = -1e30

_SC_PARAMS = pltpu.CompilerParams(use_tc_tiling_on_sc=False)


def _make_mesh():
    return plsc.VectorSubcoreMesh(
        core_axis_name="c", subcore_axis_name="s", num_cores=NC
    )


def _make_deg():
    def body(dstm, zrows, ones_h, out0, out1, degacc, dst_v, ones_v, sem):
        c = lax.axis_index("c")
        s = lax.axis_index("s")
        r0 = s * ROWS_PER_TILE
        pltpu.sync_copy(zrows, degacc.at[pl.ds(r0, ROWS_PER_TILE)])
        pltpu.sync_copy(ones_h, ones_v)
        plsc.subcore_barrier()

        def chunk(g, carry):
            rr = c * ER_PER_CORE + s * ERD_PER_TILE + g * DCH
            pltpu.sync_copy(dstm.at[pl.ds(rr, DCH)], dst_v)
            for j in range(DCH):
                pltpu.sync_copy(ones_v, degacc.at[dst_v.at[j]], add=True)
            return carry

        lax.fori_loop(0, NDCHUNK, chunk, 0)
        plsc.subcore_barrier()

        @pl.when(c == 0)
        def _():
            pltpu.sync_copy(degacc.at[pl.ds(r0, ROWS_PER_TILE)],
                            out0.at[pl.ds(r0, ROWS_PER_TILE)])

        @pl.when(c == 1)
        def _():
            pltpu.sync_copy(degacc.at[pl.ds(r0, ROWS_PER_TILE)],
                            out1.at[pl.ds(r0, ROWS_PER_TILE)])

    return pl.kernel(
        body,
        out_type=[
            jax.ShapeDtypeStruct((NP, DW), jnp.float32),
            jax.ShapeDtypeStruct((NP, DW), jnp.float32),
        ],
        mesh=_make_mesh(),
        scratch_types=[
            pltpu.VMEM_SHARED((NP, DW), jnp.float32),
            pltpu.VMEM((DCH, 128), jnp.int32),
            pltpu.VMEM((128, DW), jnp.float32),
            pltpu.SemaphoreType.DMA,
        ],
        compiler_params=_SC_PARAMS,
    )


def _make_propagate():
    def body(x0, x1, srcm, dstm, zrows,
             out0, out1,
             acc, srcb, dstb, rows0, rows1, gsem, ssem, isem):
        c = lax.axis_index("c")
        s = lax.axis_index("s")
        r0 = s * ROWS_PER_TILE
        base = s * ER_PER_TILE
        pltpu.sync_copy(zrows, acc.at[pl.ds(r0, ROWS_PER_TILE)])
        plsc.subcore_barrier()

        def run(tab):
            def srow(sig, r):
                return srcb.at[sig * BR + r]

            def drow(sig, r):
                return dstb.at[sig * BR + r]

            def fire_gathers(rows_ref, sig, ci):
                for j in range(CH):
                    pltpu.async_copy(tab.at[srow(sig, CH * ci + j)],
                                     rows_ref.at[j], gsem)

            def wait_gathers(rows_ref, sig, ci):
                for j in range(CH):
                    pltpu.make_async_copy(tab.at[srow(sig, CH * ci + j)],
                                          rows_ref.at[j], gsem).wait()

            def fire_scatters(rows_ref, sig, ci):
                for j in range(CH):
                    pltpu.async_copy(rows_ref.at[j],
                                     acc.at[drow(sig, CH * ci + j)],
                                     ssem, add=True)

            def wait_scatters(rows_ref, sig, ci):
                for j in range(CH):
                    pltpu.make_async_copy(rows_ref.at[j],
                                          acc.at[drow(sig, CH * ci + j)],
                                          ssem).wait()

            def prefetch_idx(k_next, sig_next):
                rr = base + k_next * BR
                pltpu.async_copy(srcm.at[pl.ds(rr, BR)],
                                 srcb.at[pl.ds(sig_next * BR, BR)], isem)
                pltpu.async_copy(dstm.at[pl.ds(rr, BR)],
                                 dstb.at[pl.ds(sig_next * BR, BR)], isem)

            def wait_idx(sig_next):
                pltpu.make_async_copy(srcm.at[pl.ds(base, BR)],
                                      srcb.at[pl.ds(sig_next * BR, BR)],
                                      isem).wait()
                pltpu.make_async_copy(dstm.at[pl.ds(base, BR)],
                                      dstb.at[pl.ds(sig_next * BR, BR)],
                                      isem).wait()

            def do_block(k, sig, first):
                for ci in range(BCH):
                    b = ci % 2
                    rows_b = rows0 if b == 0 else rows1
                    rows_nb = rows1 if b == 0 else rows0
                    wait_gathers(rows_b, sig, ci)
                    if ci == 0:
                        if first:
                            prefetch_idx(1, 1)
                        else:
                            wait_scatters(rows_nb, 1 - sig, BCH - 1)

                            @pl.when(k < NBLK - 1)
                            def _():
                                prefetch_idx(k + 1, 1 - sig)
                    else:
                        wait_scatters(rows_nb, sig, ci - 1)
                    if ci == BCH - 1:
                        if first:
                            wait_idx(1)
                            fire_gathers(rows_nb, 1, 0)
                        else:
                            @pl.when(k < NBLK - 1)
                            def _():
                                wait_idx(1 - sig)
                                fire_gathers(rows_nb, 1 - sig, 0)
                    else:
                        fire_gathers(rows_nb, sig, ci + 1)
                    fire_scatters(rows_b, sig, ci)

            pltpu.sync_copy(srcm.at[pl.ds(base, BR)], srcb.at[pl.ds(0, BR)])
            pltpu.sync_copy(dstm.at[pl.ds(base, BR)], dstb.at[pl.ds(0, BR)])
            fire_gathers(rows0, 0, 0)
            do_block(0, 0, True)

            def outer(k, carry):
                do_block(k, lax.rem(k, 2), False)
                return carry

            lax.fori_loop(1, NBLK, outer, 0)
            wait_scatters(rows1, 0, BCH - 1)

        @pl.when(c == 0)
        def _():
            run(x0)

        @pl.when(c == 1)
        def _():
            run(x1)

        plsc.subcore_barrier()

        @pl.when(c == 0)
        def _():
            pltpu.sync_copy(acc.at[pl.ds(r0, ROWS_PER_TILE)],
                            out0.at[pl.ds(r0, ROWS_PER_TILE)])

        @pl.when(c == 1)
        def _():
            pltpu.sync_copy(acc.at[pl.ds(r0, ROWS_PER_TILE)],
                            out1.at[pl.ds(r0, ROWS_PER_TILE)])

    return pl.kernel(
        body,
        out_type=[
            jax.ShapeDtypeStruct((NP, FH), jnp.float32),
            jax.ShapeDtypeStruct((NP, FH), jnp.float32),
        ],
        mesh=_make_mesh(),
        scratch_types=[
            pltpu.VMEM_SHARED((NP, FH), jnp.float32),
            pltpu.VMEM((2 * BR, 128), jnp.int32),
            pltpu.VMEM((2 * BR, 128), jnp.int32),
            pltpu.VMEM((CH, 128, FH), jnp.float32),
            pltpu.VMEM((CH, 128, FH), jnp.float32),
            pltpu.SemaphoreType.DMA,
            pltpu.SemaphoreType.DMA,
            pltpu.SemaphoreType.DMA,
        ],
        compiler_params=_SC_PARAMS,
    )


_deg = _make_deg()
_prop = _make_propagate()


def _divide(a0, a1, deg0, deg1):
    R = NP // 8

    def body(a0_ref, a1_ref, d0_ref, d1_ref, h0_ref, h1_ref, r_ref):
        recip = 1.0 / jnp.maximum(d0_ref[...] + d1_ref[...], 1.0)
        h0_ref[...] = a0_ref[...] * recip
        h1_ref[...] = a1_ref[...] * recip
        r_ref[...] = recip

    row = lambda i: (i, 0)
    return pl.pallas_call(
        body,
        grid=(NP // R,),
        in_specs=[
            pl.BlockSpec((R, FH), row),
            pl.BlockSpec((R, FH), row),
            pl.BlockSpec((R, 1), row),
            pl.BlockSpec((R, 1), row),
        ],
        out_specs=[
            pl.BlockSpec((R, FH), row),
            pl.BlockSpec((R, FH), row),
            pl.BlockSpec((R, 1), row),
        ],
        out_shape=[
            jax.ShapeDtypeStruct((NP, FH), jnp.float32),
            jax.ShapeDtypeStruct((NP, FH), jnp.float32),
            jax.ShapeDtypeStruct((NP, 1), jnp.float32),
        ],
    )(a0, a1, deg0, deg1)


def _scores(x0i, h0i, b0i, x1i, h1i, b1i, recipi,
            xu0, hu0, bu0, xu1, hu1, bu1, recipu):
    R = 4000

    def body(x0r, h0r, b0r, x1r, h1r, b1r, rr,
             xu0r, hu0r, bu0r, xu1r, hu1r, bu1r, rur, out):
        ru = rr[...]
        u0 = (xu0r[...] + hu0r[...] + bu0r[...] * rur[...]) / 3.0
        u1 = (xu1r[...] + hu1r[...] + bu1r[...] * rur[...]) / 3.0
        f0 = (x0r[...] + h0r[...] + b0r[...] * ru) / 3.0
        f1 = (x1r[...] + h1r[...] + b1r[...] * ru) / 3.0
        out[...] = jnp.sum(f0 * u0 + f1 * u1, axis=1, keepdims=True)

    row = lambda i: (i, 0)
    rep = lambda i: (0, 0)
    return pl.pallas_call(
        body,
        grid=(NUM_ITEMS // R,),
        in_specs=[pl.BlockSpec((R, FH), row)] * 6
        + [pl.BlockSpec((R, 1), row)]
        + [pl.BlockSpec((1, FH), rep)] * 6
        + [pl.BlockSpec((1, 1), rep)],
        out_specs=pl.BlockSpec((R, 1), row),
        out_shape=jax.ShapeDtypeStruct((NUM_ITEMS, 1), jnp.float32),
    )(x0i, h0i, b0i, x1i, h1i, b1i, recipi, xu0, hu0, bu0, xu1, hu1, bu1, recipu)


def _topk(s2d):
    def body(s_ref, vals_ref, idx_ref):
        s = s_ref[...]
        ridx = lax.broadcasted_iota(jnp.int32, s.shape, 0)
        cidx = lax.broadcasted_iota(jnp.int32, s.shape, 1)
        flat = ridx * 128 + cidx
        lane = lax.broadcasted_iota(jnp.int32, (1, 128), 1)
        vals = jnp.zeros((1, 128), jnp.float32)
        idxs = jnp.zeros((1, 128), jnp.int32)
        for k in range(TOPK):
            m = jnp.max(s)
            j = jnp.min(jnp.where(s == m, flat, jnp.int32(2**31 - 1)))
            vals = jnp.where(lane == k, m, vals)
            idxs = jnp.where(lane == k, j, idxs)
            s = jnp.where(flat == j, NEG, s)
        vals_ref[...] = vals
        idx_ref[...] = idxs

    return pl.pallas_call(
        body,
        out_shape=[
            jax.ShapeDtypeStruct((1, 128), jnp.float32),
            jax.ShapeDtypeStruct((1, 128), jnp.int32),
        ],
    )(s2d)


def kernel(user_table, item_table, edge_index, user_id, topk):
    del topk
    x = jnp.concatenate([user_table, item_table], axis=0)
    x0 = jnp.pad(x[:, :FH], ((0, NP - N), (0, 0)))
    x1 = jnp.pad(x[:, FH:], ((0, NP - N), (0, 0)))
    pad_e = jnp.full((EP - E,), NP - 1, dtype=jnp.int32)
    srcm = jnp.concatenate([edge_index[0], pad_e]).reshape(ER, 128)
    dstm = jnp.concatenate([edge_index[1], pad_e]).reshape(ER, 128)
    zrows = jnp.zeros((ROWS_PER_TILE, FH), jnp.float32)
    zdeg = jnp.zeros((ROWS_PER_TILE, DW), jnp.float32)
    ones = jnp.ones((128, DW), jnp.float32)

    deg0, deg1 = _deg(dstm, zdeg, ones)
    a0, a1 = _prop(x0, x1, srcm, dstm, zrows)
    h0, h1, recip = _divide(a0, a1, deg0[:, :1], deg1[:, :1])
    b0, b1 = _prop(h0, h1, srcm, dstm, zrows)

    uid = jnp.asarray(user_id, jnp.int32)
    sl = lambda arr: lax.dynamic_slice_in_dim(arr, uid, 1)
    scores = _scores(
        x0[NUM_USERS:N], h0[NUM_USERS:N], b0[NUM_USERS:N],
        x1[NUM_USERS:N], h1[NUM_USERS:N], b1[NUM_USERS:N], recip[NUM_USERS:N],
        sl(x0), sl(h0), sl(b0), sl(x1), sl(h1), sl(b1), sl(recip),
    )
    s2d = jnp.pad(scores[:, 0], (0, 40064 - NUM_ITEMS),
                  constant_values=NEG).reshape(313, 128)
    vals, idx = _topk(s2d)
    return vals[0, :TOPK], idx[0, :TOPK]


# speedup vs baseline: 9.8533x; 1.0719x over previous
import jax
import jax.numpy as jnp
from jax import lax
from jax.experimental import pallas as pl
from jax.experimental.pallas import tpu as pltpu
from jax.experimental.pallas import tpu_sc as plsc

NUM_USERS = 10000
NUM_ITEMS = 40000
F = 64
FH = 32
N = NUM_USERS + NUM_ITEMS
E = 800000
NC = 2
NS = 16
NP = 50048
ROWS_PER_TILE = NP // NS
EP = 802816
ER = EP // 128
ER_PER_TILE = ER // NS
CH = 2
CHR = CH * 128
NCHUNK = ER_PER_TILE // CH
BCH = 4
NBLK = NCHUNK // BCH
BR = BCH * CH
DW = 8
ER_PER_CORE = ER // NC
ERD_PER_TILE = ER_PER_CORE // NS
DCH = 4
NDCHUNK = ERD_PER_TILE // DCH
TOPK = 10
NEG = -1e30

_SC_PARAMS = pltpu.CompilerParams(use_tc_tiling_on_sc=False)


def _make_mesh():
    return plsc.VectorSubcoreMesh(
        core_axis_name="c", subcore_axis_name="s", num_cores=NC
    )


def _make_deg():
    def body(dstm, zrows, ones_h, out0, out1, degacc, dst_v, ones_v, sem):
        c = lax.axis_index("c")
        s = lax.axis_index("s")
        r0 = s * ROWS_PER_TILE
        pltpu.sync_copy(zrows, degacc.at[pl.ds(r0, ROWS_PER_TILE)])
        pltpu.sync_copy(ones_h, ones_v)
        plsc.subcore_barrier()

        def chunk(g, carry):
            rr = c * ER_PER_CORE + s * ERD_PER_TILE + g * DCH
            pltpu.sync_copy(dstm.at[pl.ds(rr, DCH)], dst_v)
            for j in range(DCH):
                pltpu.sync_copy(ones_v, degacc.at[dst_v.at[j]], add=True)
            return carry

        lax.fori_loop(0, NDCHUNK, chunk, 0)
        plsc.subcore_barrier()

        @pl.when(c == 0)
        def _():
            pltpu.sync_copy(degacc.at[pl.ds(r0, ROWS_PER_TILE)],
                            out0.at[pl.ds(r0, ROWS_PER_TILE)])

        @pl.when(c == 1)
        def _():
            pltpu.sync_copy(degacc.at[pl.ds(r0, ROWS_PER_TILE)],
                            out1.at[pl.ds(r0, ROWS_PER_TILE)])

    return pl.kernel(
        body,
        out_type=[
            jax.ShapeDtypeStruct((NP, DW), jnp.float32),
            jax.ShapeDtypeStruct((NP, DW), jnp.float32),
        ],
        mesh=_make_mesh(),
        scratch_types=[
            pltpu.VMEM_SHARED((NP, DW), jnp.float32),
            pltpu.VMEM((DCH, 128), jnp.int32),
            pltpu.VMEM((128, DW), jnp.float32),
            pltpu.SemaphoreType.DMA,
        ],
        compiler_params=_SC_PARAMS,
    )


def _make_propagate(tab_rows, off1):
    def body(tab, srcm, dstm, zrows, out,
             acc, srcb, dstb, rows0, rows1, gsem, ssem, isem):
        c = lax.axis_index("c")
        s = lax.axis_index("s")
        r0 = s * ROWS_PER_TILE
        base = s * ER_PER_TILE
        pltpu.sync_copy(zrows, acc.at[pl.ds(r0, ROWS_PER_TILE)])
        plsc.subcore_barrier()

        def run(tabv):
            def srow(sig, r):
                return srcb.at[sig * BR + r]

            def drow(sig, r):
                return dstb.at[sig * BR + r]

            def fire_gathers(rows_ref, sig, ci):
                for j in range(CH):
                    pltpu.async_copy(tabv.at[srow(sig, CH * ci + j)],
                                     rows_ref.at[pl.ds(j * 128, 128)], gsem)

            def wait_gathers(rows_ref):
                pltpu.make_async_copy(tabv.at[pl.ds(0, CHR)],
                                      rows_ref, gsem).wait()

            def fire_scatters(rows_ref, sig, ci):
                for j in range(CH):
                    pltpu.async_copy(rows_ref.at[pl.ds(j * 128, 128)],
                                     acc.at[drow(sig, CH * ci + j)],
                                     ssem, add=True)

            def wait_scatters(rows_ref):
                pltpu.make_async_copy(rows_ref, acc.at[pl.ds(0, CHR)],
                                      ssem).wait()

            def prefetch_idx(k_next, sig_next):
                rr = base + k_next * BR
                pltpu.async_copy(srcm.at[pl.ds(rr, BR)],
                                 srcb.at[pl.ds(sig_next * BR, BR)], isem)
                pltpu.async_copy(dstm.at[pl.ds(rr, BR)],
                                 dstb.at[pl.ds(sig_next * BR, BR)], isem)

            def wait_idx():
                pltpu.make_async_copy(srcm.at[pl.ds(0, 2 * BR)],
                                      srcb, isem).wait()

            def do_block(k, sig, first):
                for ci in range(BCH):
                    b = ci % 2
                    rows_b = rows0 if b == 0 else rows1
                    rows_nb = rows1 if b == 0 else rows0
                    wait_gathers(rows_b)
                    if ci == 0:
                        if first:
                            prefetch_idx(1, 1)
                        else:
                            wait_scatters(rows_nb)

                            @pl.when(k < NBLK - 1)
                            def _():
                                prefetch_idx(k + 1, 1 - sig)
                    else:
                        wait_scatters(rows_nb)
                    if ci == BCH - 1:
                        if first:
                            wait_idx()
                            fire_gathers(rows_nb, 1, 0)
                        else:
                            @pl.when(k < NBLK - 1)
                            def _():
                                wait_idx()
                                fire_gathers(rows_nb, 1 - sig, 0)
                    else:
                        fire_gathers(rows_nb, sig, ci + 1)
                    fire_scatters(rows_b, sig, ci)

            pltpu.sync_copy(srcm.at[pl.ds(base, BR)], srcb.at[pl.ds(0, BR)])
            pltpu.sync_copy(dstm.at[pl.ds(base, BR)], dstb.at[pl.ds(0, BR)])
            fire_gathers(rows0, 0, 0)
            do_block(0, 0, True)

            def outer(k, carry):
                do_block(k, lax.rem(k, 2), False)
                return carry

            lax.fori_loop(1, NBLK, outer, 0)
            wait_scatters(rows1)

        @pl.when(c == 0)
        def _():
            run(tab)

        @pl.when(c == 1)
        def _():
            run(tab.at[pl.ds(off1, tab_rows - off1)])

        plsc.subcore_barrier()

        @pl.when(c == 0)
        def _():
            pltpu.sync_copy(acc.at[pl.ds(r0, ROWS_PER_TILE)],
                            out.at[0].at[pl.ds(r0, ROWS_PER_TILE)])

        @pl.when(c == 1)
        def _():
            pltpu.sync_copy(acc.at[pl.ds(r0, ROWS_PER_TILE)],
                            out.at[1].at[pl.ds(r0, ROWS_PER_TILE)])

    return pl.kernel(
        body,
        out_type=[jax.ShapeDtypeStruct((NC, NP, FH), jnp.float32)],
        mesh=_make_mesh(),
        scratch_types=[
            pltpu.VMEM_SHARED((NP, FH), jnp.float32),
            pltpu.VMEM((2 * BR, 128), jnp.int32),
            pltpu.VMEM((2 * BR, 128), jnp.int32),
            pltpu.VMEM((CHR, FH), jnp.float32),
            pltpu.VMEM((CHR, FH), jnp.float32),
            pltpu.SemaphoreType.DMA,
            pltpu.SemaphoreType.DMA,
            pltpu.SemaphoreType.DMA,
        ],
        compiler_params=_SC_PARAMS,
    )


_deg = _make_deg()
_prop1 = _make_propagate(2 * N, 1)
_prop2 = _make_propagate(NC * NP, NP)


def _divide(a, deg0, deg1):
    R = NP // 8

    def body(a_ref, d0_ref, d1_ref, h_ref):
        recip = 1.0 / jnp.maximum(d0_ref[...] + d1_ref[...], 1.0)
        h_ref[...] = a_ref[...] * recip

    return pl.pallas_call(
        body,
        grid=(NC, NP // R),
        in_specs=[
            pl.BlockSpec((1, R, FH), lambda c, i: (c, i, 0)),
            pl.BlockSpec((R, 1), lambda c, i: (i, 0)),
            pl.BlockSpec((R, 1), lambda c, i: (i, 0)),
        ],
        out_specs=pl.BlockSpec((1, R, FH), lambda c, i: (c, i, 0)),
        out_shape=jax.ShapeDtypeStruct((NC, NP, FH), jnp.float32),
    )(a, deg0, deg1)


def _scores(xcat, h, b, deg0, deg1, xu, hu, bu, d0u, d1u):
    R = 2000
    OFF = NUM_USERS // R

    def body(x_r, h0_r, h1_r, b0_r, b1_r, d0_r, d1_r,
             xu_r, hu_r, bu_r, d0u_r, d1u_r, out):
        recip = 1.0 / jnp.maximum(d0_r[...] + d1_r[...], 1.0)
        ru = 1.0 / jnp.maximum(d0u_r[...] + d1u_r[...], 1.0)
        xu_b = xu_r[...]
        u0 = (xu_b[:, :FH] + hu_r[0] + bu_r[0] * ru) / 3.0
        u1 = (xu_b[:, FH:] + hu_r[1] + bu_r[1] * ru) / 3.0
        x_b = x_r[...]
        f0 = (x_b[:, :FH] + h0_r[0] + b0_r[0] * recip) / 3.0
        f1 = (x_b[:, FH:] + h1_r[0] + b1_r[0] * recip) / 3.0
        bf = lambda v: v.astype(jnp.bfloat16).astype(jnp.float32)
        t = bf(f0) * bf(u0) + bf(f1) * bf(u1)
        w = FH
        while w > 1:
            w //= 2
            t = t[:, :w] + t[:, w:]
        out[...] = t

    itm = lambda i: (i + OFF, 0)
    itm3a = lambda i: (0, i + OFF, 0)
    itm3b = lambda i: (1, i + OFF, 0)
    rep = lambda i: (0, 0)
    rep3 = lambda i: (0, 0, 0)
    return pl.pallas_call(
        body,
        grid=(NUM_ITEMS // R,),
        in_specs=[
            pl.BlockSpec((R, F), itm),
            pl.BlockSpec((1, R, FH), itm3a),
            pl.BlockSpec((1, R, FH), itm3b),
            pl.BlockSpec((1, R, FH), itm3a),
            pl.BlockSpec((1, R, FH), itm3b),
            pl.BlockSpec((R, 1), itm),
            pl.BlockSpec((R, 1), itm),
            pl.BlockSpec((1, F), rep),
            pl.BlockSpec((NC, 1, FH), rep3),
            pl.BlockSpec((NC, 1, FH), rep3),
            pl.BlockSpec((1, 1), rep),
            pl.BlockSpec((1, 1), rep),
        ],
        out_specs=pl.BlockSpec((R, 1), lambda i: (i, 0)),
        out_shape=jax.ShapeDtypeStruct((NUM_ITEMS, 1), jnp.float32),
    )(xcat, h, h, b, b, deg0, deg1, xu, hu, bu, d0u, d1u)


def _topk(s2d):
    def body(s_ref, vals_ref, idx_ref):
        s = s_ref[...]
        ridx = lax.broadcasted_iota(jnp.int32, s.shape, 0)
        cidx = lax.broadcasted_iota(jnp.int32, s.shape, 1)
        flat = ridx * 128 + cidx
        lane = lax.broadcasted_iota(jnp.int32, (1, 128), 1)
        vals = jnp.zeros((1, 128), jnp.float32)
        idxs = jnp.zeros((1, 128), jnp.int32)
        for k in range(TOPK):
            m = jnp.max(s)
            j = jnp.min(jnp.where(s == m, flat, jnp.int32(2**31 - 1)))
            vals = jnp.where(lane == k, m, vals)
            idxs = jnp.where(lane == k, j, idxs)
            s = jnp.where(flat == j, NEG, s)
        vals_ref[...] = vals
        idx_ref[...] = idxs

    return pl.pallas_call(
        body,
        out_shape=[
            jax.ShapeDtypeStruct((1, 128), jnp.float32),
            jax.ShapeDtypeStruct((1, 128), jnp.int32),
        ],
    )(s2d)


def kernel(user_table, item_table, edge_index, user_id, topk):
    del topk
    xcat = jnp.concatenate([user_table, item_table], axis=0)
    tab1 = xcat.reshape(2 * N, FH)
    src_pad = jnp.zeros((EP - E,), dtype=jnp.int32)
    srcp = jnp.concatenate([edge_index[0], src_pad])
    s1m = (srcp * 2).reshape(ER, 128)
    s2m = srcp.reshape(ER, 128)
    dstm = jnp.concatenate(
        [edge_index[1], jnp.full((EP - E,), NP - 1, dtype=jnp.int32)]
    ).reshape(ER, 128)
    zrows = jnp.zeros((ROWS_PER_TILE, FH), jnp.float32)
    zdeg = jnp.zeros((ROWS_PER_TILE, DW), jnp.float32)
    ones = jnp.ones((128, DW), jnp.float32)

    deg0, deg1 = _deg(dstm, zdeg, ones)
    (a,) = _prop1(tab1, s1m, dstm, zrows)
    h = _divide(a, deg0[:, :1], deg1[:, :1])
    (b,) = _prop2(h.reshape(NC * NP, FH), s2m, dstm, zrows)

    uid = jnp.asarray(user_id, jnp.int32)
    xu = lax.dynamic_slice(xcat, (uid, 0), (1, F))
    hu = lax.dynamic_slice(h, (0, uid, 0), (NC, 1, FH))
    bu = lax.dynamic_slice(b, (0, uid, 0), (NC, 1, FH))
    d0u = lax.dynamic_slice(deg0, (uid, 0), (1, 1))
    d1u = lax.dynamic_slice(deg1, (uid, 0), (1, 1))
    scores = _scores(xcat, h, b, deg0[:, :1], deg1[:, :1], xu, hu, bu, d0u, d1u)
    s2d = jnp.pad(scores[:, 0], (0, 40064 - NUM_ITEMS),
                  constant_values=NEG).reshape(313, 128)
    vals, idx = _topk(s2d)
    return vals[0, :TOPK], idx[0, :TOPK]


# speedup vs baseline: 10.0145x; 1.0164x over previous
import jax
import jax.numpy as jnp
from jax import lax
from jax.experimental import pallas as pl
from jax.experimental.pallas import tpu as pltpu
from jax.experimental.pallas import tpu_sc as plsc

NUM_USERS = 10000
NUM_ITEMS = 40000
F = 64
FH = 32
N = NUM_USERS + NUM_ITEMS
E = 800000
NC = 2
NS = 16
NP = 50048
ROWS_PER_TILE = NP // NS
EP = 802816
ER = EP // 128
ER_PER_TILE = ER // NS
CH = 2
CHR = CH * 128
NCHUNK = ER_PER_TILE // CH
BCH = 4
NBLK = NCHUNK // BCH
BR = BCH * CH
DW = 8
ER_PER_CORE = ER // NC
ERD_PER_TILE = ER_PER_CORE // NS
DCH = 4
NDCHUNK = ERD_PER_TILE // DCH
TOPK = 10
NEG = -1e30

_SC_PARAMS = pltpu.CompilerParams(use_tc_tiling_on_sc=False)


def _make_mesh():
    return plsc.VectorSubcoreMesh(
        core_axis_name="c", subcore_axis_name="s", num_cores=NC
    )


def _make_deg():
    NDBLK = ERD_PER_TILE // DCH

    def body(dstm, zrows, ones_h, out0, out1, degacc, dstb, ones_v, ssem, isem):
        c = lax.axis_index("c")
        s = lax.axis_index("s")
        r0 = s * ROWS_PER_TILE
        base = c * ER_PER_CORE + s * ERD_PER_TILE
        pltpu.sync_copy(zrows, degacc.at[pl.ds(r0, ROWS_PER_TILE)])
        pltpu.sync_copy(ones_h, ones_v)
        plsc.subcore_barrier()

        def prefetch(k_next, sig_next):
            pltpu.async_copy(dstm.at[pl.ds(base + k_next * DCH, DCH)],
                             dstb.at[pl.ds(sig_next * DCH, DCH)], isem)

        def wait_idx(sig):
            pltpu.make_async_copy(dstm.at[pl.ds(base, DCH)],
                                  dstb.at[pl.ds(sig * DCH, DCH)], isem).wait()

        def fire(sig):
            for j in range(DCH):
                pltpu.async_copy(ones_v.at[pl.ds(j * 128, 128)],
                                 degacc.at[dstb.at[sig * DCH + j]],
                                 ssem, add=True)

        def drain():
            pltpu.make_async_copy(ones_v, degacc.at[pl.ds(0, DCH * 128)],
                                  ssem).wait()

        pltpu.sync_copy(dstm.at[pl.ds(base, DCH)], dstb.at[pl.ds(0, DCH)])
        prefetch(1, 1)
        fire(0)

        def outer(k, carry):
            sig = lax.rem(k, 4)
            wait_idx(sig)

            @pl.when(k >= 2)
            def _():
                drain()

            @pl.when(k < NDBLK - 1)
            def _():
                prefetch(k + 1, lax.rem(k + 1, 4))

            fire(sig)
            return carry

        lax.fori_loop(1, NDBLK, outer, 0)
        drain()
        drain()
        plsc.subcore_barrier()

        @pl.when(c == 0)
        def _():
            pltpu.sync_copy(degacc.at[pl.ds(r0, ROWS_PER_TILE)],
                            out0.at[pl.ds(r0, ROWS_PER_TILE)])

        @pl.when(c == 1)
        def _():
            pltpu.sync_copy(degacc.at[pl.ds(r0, ROWS_PER_TILE)],
                            out1.at[pl.ds(r0, ROWS_PER_TILE)])

    return pl.kernel(
        body,
        out_type=[
            jax.ShapeDtypeStruct((NP, DW), jnp.float32),
            jax.ShapeDtypeStruct((NP, DW), jnp.float32),
        ],
        mesh=_make_mesh(),
        scratch_types=[
            pltpu.VMEM_SHARED((NP, DW), jnp.float32),
            pltpu.VMEM((4 * DCH, 128), jnp.int32),
            pltpu.VMEM((DCH * 128, DW), jnp.float32),
            pltpu.SemaphoreType.DMA,
            pltpu.SemaphoreType.DMA,
        ],
        compiler_params=_SC_PARAMS,
    )


def _make_propagate(tab_rows, off1):
    def body(tab, srcm, dstm, zrows, out,
             acc, srcb, dstb, rows0, rows1, gsem, ssem, isem):
        c = lax.axis_index("c")
        s = lax.axis_index("s")
        r0 = s * ROWS_PER_TILE
        base = s * ER_PER_TILE
        pltpu.sync_copy(zrows, acc.at[pl.ds(r0, ROWS_PER_TILE)])
        plsc.subcore_barrier()

        def run(tabv):
            def srow(sig, r):
                return srcb.at[sig * BR + r]

            def drow(sig, r):
                return dstb.at[sig * BR + r]

            def fire_gathers(rows_ref, sig, ci):
                for j in range(CH):
                    pltpu.async_copy(tabv.at[srow(sig, CH * ci + j)],
                                     rows_ref.at[pl.ds(j * 128, 128)], gsem)

            def wait_gathers(rows_ref):
                pltpu.make_async_copy(tabv.at[pl.ds(0, CHR)],
                                      rows_ref, gsem).wait()

            def fire_scatters(rows_ref, sig, ci):
                for j in range(CH):
                    pltpu.async_copy(rows_ref.at[pl.ds(j * 128, 128)],
                                     acc.at[drow(sig, CH * ci + j)],
                                     ssem, add=True)

            def wait_scatters(rows_ref):
                pltpu.make_async_copy(rows_ref, acc.at[pl.ds(0, CHR)],
                                      ssem).wait()

            def prefetch_idx(k_next, sig_next):
                rr = base + k_next * BR
                pltpu.async_copy(srcm.at[pl.ds(rr, BR)],
                                 srcb.at[pl.ds(sig_next * BR, BR)], isem)
                pltpu.async_copy(dstm.at[pl.ds(rr, BR)],
                                 dstb.at[pl.ds(sig_next * BR, BR)], isem)

            def wait_idx():
                pltpu.make_async_copy(srcm.at[pl.ds(0, 2 * BR)],
                                      srcb, isem).wait()

            def do_block(k, sig, first):
                for ci in range(BCH):
                    b = ci % 2
                    rows_b = rows0 if b == 0 else rows1
                    rows_nb = rows1 if b == 0 else rows0
                    wait_gathers(rows_b)
                    if ci == 0:
                        if first:
                            prefetch_idx(1, 1)
                        else:
                            wait_scatters(rows_nb)

                            @pl.when(k < NBLK - 1)
                            def _():
                                prefetch_idx(k + 1, 1 - sig)
                    else:
                        wait_scatters(rows_nb)
                    if ci == BCH - 1:
                        if first:
                            wait_idx()
                            fire_gathers(rows_nb, 1, 0)
                        else:
                            @pl.when(k < NBLK - 1)
                            def _():
                                wait_idx()
                                fire_gathers(rows_nb, 1 - sig, 0)
                    else:
                        fire_gathers(rows_nb, sig, ci + 1)
                    fire_scatters(rows_b, sig, ci)

            pltpu.sync_copy(srcm.at[pl.ds(base, BR)], srcb.at[pl.ds(0, BR)])
            pltpu.sync_copy(dstm.at[pl.ds(base, BR)], dstb.at[pl.ds(0, BR)])
            fire_gathers(rows0, 0, 0)
            do_block(0, 0, True)

            def outer(k, carry):
                do_block(k, lax.rem(k, 2), False)
                return carry

            lax.fori_loop(1, NBLK, outer, 0)
            wait_scatters(rows1)

        @pl.when(c == 0)
        def _():
            run(tab)

        @pl.when(c == 1)
        def _():
            run(tab.at[pl.ds(off1, tab_rows - off1)])

        plsc.subcore_barrier()

        @pl.when(c == 0)
        def _():
            pltpu.sync_copy(acc.at[pl.ds(r0, ROWS_PER_TILE)],
                            out.at[0].at[pl.ds(r0, ROWS_PER_TILE)])

        @pl.when(c == 1)
        def _():
            pltpu.sync_copy(acc.at[pl.ds(r0, ROWS_PER_TILE)],
                            out.at[1].at[pl.ds(r0, ROWS_PER_TILE)])

    return pl.kernel(
        body,
        out_type=[jax.ShapeDtypeStruct((NC, NP, FH), jnp.float32)],
        mesh=_make_mesh(),
        scratch_types=[
            pltpu.VMEM_SHARED((NP, FH), jnp.float32),
            pltpu.VMEM((2 * BR, 128), jnp.int32),
            pltpu.VMEM((2 * BR, 128), jnp.int32),
            pltpu.VMEM((CHR, FH), jnp.float32),
            pltpu.VMEM((CHR, FH), jnp.float32),
            pltpu.SemaphoreType.DMA,
            pltpu.SemaphoreType.DMA,
            pltpu.SemaphoreType.DMA,
        ],
        compiler_params=_SC_PARAMS,
    )


_deg = _make_deg()
_prop1 = _make_propagate(2 * N, 1)
_prop2 = _make_propagate(NC * NP, NP)


def _divide(a, deg0, deg1):
    R = NP // 8

    def body(a_ref, d0_ref, d1_ref, h_ref):
        recip = 1.0 / jnp.maximum(d0_ref[...] + d1_ref[...], 1.0)
        h_ref[...] = a_ref[...] * recip

    return pl.pallas_call(
        body,
        grid=(NC, NP // R),
        in_specs=[
            pl.BlockSpec((1, R, FH), lambda c, i: (c, i, 0)),
            pl.BlockSpec((R, 1), lambda c, i: (i, 0)),
            pl.BlockSpec((R, 1), lambda c, i: (i, 0)),
        ],
        out_specs=pl.BlockSpec((1, R, FH), lambda c, i: (c, i, 0)),
        out_shape=jax.ShapeDtypeStruct((NC, NP, FH), jnp.float32),
    )(a, deg0, deg1)


def _scores(xcat, h, b, deg0, deg1, xu, hu, bu, d0u, d1u):
    R = 2000
    OFF = NUM_USERS // R

    def body(x_r, h0_r, h1_r, b0_r, b1_r, d0_r, d1_r,
             xu_r, hu_r, bu_r, d0u_r, d1u_r, out):
        recip = 1.0 / jnp.maximum(d0_r[...] + d1_r[...], 1.0)
        ru = 1.0 / jnp.maximum(d0u_r[...] + d1u_r[...], 1.0)
        xu_b = xu_r[...]
        u0 = (xu_b[:, :FH] + hu_r[0] + bu_r[0] * ru) / 3.0
        u1 = (xu_b[:, FH:] + hu_r[1] + bu_r[1] * ru) / 3.0
        x_b = x_r[...]
        f0 = (x_b[:, :FH] + h0_r[0] + b0_r[0] * recip) / 3.0
        f1 = (x_b[:, FH:] + h1_r[0] + b1_r[0] * recip) / 3.0
        bf = lambda v: v.astype(jnp.bfloat16).astype(jnp.float32)
        t = bf(f0) * bf(u0) + bf(f1) * bf(u1)
        w = FH
        while w > 1:
            w //= 2
            t = t[:, :w] + t[:, w:]
        out[...] = t

    itm = lambda i: (i + OFF, 0)
    itm3a = lambda i: (0, i + OFF, 0)
    itm3b = lambda i: (1, i + OFF, 0)
    rep = lambda i: (0, 0)
    rep3 = lambda i: (0, 0, 0)
    return pl.pallas_call(
        body,
        grid=(NUM_ITEMS // R,),
        in_specs=[
            pl.BlockSpec((R, F), itm),
            pl.BlockSpec((1, R, FH), itm3a),
            pl.BlockSpec((1, R, FH), itm3b),
            pl.BlockSpec((1, R, FH), itm3a),
            pl.BlockSpec((1, R, FH), itm3b),
            pl.BlockSpec((R, 1), itm),
            pl.BlockSpec((R, 1), itm),
            pl.BlockSpec((1, F), rep),
            pl.BlockSpec((NC, 1, FH), rep3),
            pl.BlockSpec((NC, 1, FH), rep3),
            pl.BlockSpec((1, 1), rep),
            pl.BlockSpec((1, 1), rep),
        ],
        out_specs=pl.BlockSpec((R, 1), lambda i: (i, 0)),
        out_shape=jax.ShapeDtypeStruct((NUM_ITEMS, 1), jnp.float32),
    )(xcat, h, h, b, b, deg0, deg1, xu, hu, bu, d0u, d1u)


def _topk(s2d):
    def body(s_ref, vals_ref, idx_ref):
        s = s_ref[...]
        ridx = lax.broadcasted_iota(jnp.int32, s.shape, 0)
        cidx = lax.broadcasted_iota(jnp.int32, s.shape, 1)
        flat = ridx * 128 + cidx
        lane = lax.broadcasted_iota(jnp.int32, (1, 128), 1)
        vals = jnp.zeros((1, 128), jnp.float32)
        idxs = jnp.zeros((1, 128), jnp.int32)
        for k in range(TOPK):
            m = jnp.max(s)
            j = jnp.min(jnp.where(s == m, flat, jnp.int32(2**31 - 1)))
            vals = jnp.where(lane == k, m, vals)
            idxs = jnp.where(lane == k, j, idxs)
            s = jnp.where(flat == j, NEG, s)
        vals_ref[...] = vals
        idx_ref[...] = idxs

    return pl.pallas_call(
        body,
        out_shape=[
            jax.ShapeDtypeStruct((1, 128), jnp.float32),
            jax.ShapeDtypeStruct((1, 128), jnp.int32),
        ],
    )(s2d)


def kernel(user_table, item_table, edge_index, user_id, topk):
    del topk
    xcat = jnp.concatenate([user_table, item_table], axis=0)
    tab1 = xcat.reshape(2 * N, FH)
    src_pad = jnp.zeros((EP - E,), dtype=jnp.int32)
    srcp = jnp.concatenate([edge_index[0], src_pad])
    s1m = (srcp * 2).reshape(ER, 128)
    s2m = srcp.reshape(ER, 128)
    dstm = jnp.concatenate(
        [edge_index[1], jnp.full((EP - E,), NP - 1, dtype=jnp.int32)]
    ).reshape(ER, 128)
    zrows = jnp.zeros((ROWS_PER_TILE, FH), jnp.float32)
    zdeg = jnp.zeros((ROWS_PER_TILE, DW), jnp.float32)
    ones = jnp.ones((DCH * 128, DW), jnp.float32)

    deg0, deg1 = _deg(dstm, zdeg, ones)
    (a,) = _prop1(tab1, s1m, dstm, zrows)
    h = _divide(a, deg0[:, :1], deg1[:, :1])
    (b,) = _prop2(h.reshape(NC * NP, FH), s2m, dstm, zrows)

    uid = jnp.asarray(user_id, jnp.int32)
    xu = lax.dynamic_slice(xcat, (uid, 0), (1, F))
    hu = lax.dynamic_slice(h, (0, uid, 0), (NC, 1, FH))
    bu = lax.dynamic_slice(b, (0, uid, 0), (NC, 1, FH))
    d0u = lax.dynamic_slice(deg0, (uid, 0), (1, 1))
    d1u = lax.dynamic_slice(deg1, (uid, 0), (1, 1))
    scores = _scores(xcat, h, b, deg0[:, :1], deg1[:, :1], xu, hu, bu, d0u, d1u)
    s2d = jnp.pad(scores[:, 0], (0, 40064 - NUM_ITEMS),
                  constant_values=NEG).reshape(313, 128)
    vals, idx = _topk(s2d)
    return vals[0, :TOPK], idx[0, :TOPK]


# speedup vs baseline: 10.0255x; 1.0011x over previous
import jax
import jax.numpy as jnp
from jax import lax
from jax.experimental import pallas as pl
from jax.experimental.pallas import tpu as pltpu
from jax.experimental.pallas import tpu_sc as plsc

NUM_USERS = 10000
NUM_ITEMS = 40000
F = 64
FH = 32
N = NUM_USERS + NUM_ITEMS
E = 800000
NC = 2
NS = 16
NP = 50048
ROWS_PER_TILE = NP // NS
EP = 802816
ER = EP // 128
ER_PER_TILE = ER // NS
CH = 2
CHR = CH * 128
NCHUNK = ER_PER_TILE // CH
BCH = 4
NBLK = NCHUNK // BCH
BR = BCH * CH
DW = 8
ER_PER_CORE = ER // NC
ERD_PER_TILE = ER_PER_CORE // NS
DCH = 4
NDCHUNK = ERD_PER_TILE // DCH
TOPK = 10
NEG = -1e30

_SC_PARAMS = pltpu.CompilerParams(use_tc_tiling_on_sc=False)


def _make_mesh():
    return plsc.VectorSubcoreMesh(
        core_axis_name="c", subcore_axis_name="s", num_cores=NC
    )


def _make_deg():
    NDBLK = ERD_PER_TILE // DCH

    def body(dstm, zrows, ones_h, out0, out1, degacc, dstb, ones_v, ssem, isem):
        c = lax.axis_index("c")
        s = lax.axis_index("s")
        r0 = s * ROWS_PER_TILE
        base = c * ER_PER_CORE + s * ERD_PER_TILE
        pltpu.sync_copy(zrows, degacc.at[pl.ds(r0, ROWS_PER_TILE)])
        pltpu.sync_copy(ones_h, ones_v)
        plsc.subcore_barrier()

        def prefetch(k_next, sig_next):
            pltpu.async_copy(dstm.at[pl.ds(base + k_next * DCH, DCH)],
                             dstb.at[pl.ds(sig_next * DCH, DCH)], isem)

        def wait_idx(sig):
            pltpu.make_async_copy(dstm.at[pl.ds(base, DCH)],
                                  dstb.at[pl.ds(sig * DCH, DCH)], isem).wait()

        def fire(sig):
            for j in range(DCH):
                pltpu.async_copy(ones_v.at[pl.ds(j * 128, 128)],
                                 degacc.at[dstb.at[sig * DCH + j]],
                                 ssem, add=True)

        def drain():
            pltpu.make_async_copy(ones_v, degacc.at[pl.ds(0, DCH * 128)],
                                  ssem).wait()

        pltpu.sync_copy(dstm.at[pl.ds(base, DCH)], dstb.at[pl.ds(0, DCH)])
        prefetch(1, 1)
        fire(0)

        def outer(k, carry):
            sig = lax.rem(k, 4)
            wait_idx(sig)

            @pl.when(k >= 2)
            def _():
                drain()

            @pl.when(k < NDBLK - 1)
            def _():
                prefetch(k + 1, lax.rem(k + 1, 4))

            fire(sig)
            return carry

        lax.fori_loop(1, NDBLK, outer, 0)
        drain()
        drain()
        plsc.subcore_barrier()

        @pl.when(c == 0)
        def _():
            pltpu.sync_copy(degacc.at[pl.ds(r0, ROWS_PER_TILE)],
                            out0.at[pl.ds(r0, ROWS_PER_TILE)])

        @pl.when(c == 1)
        def _():
            pltpu.sync_copy(degacc.at[pl.ds(r0, ROWS_PER_TILE)],
                            out1.at[pl.ds(r0, ROWS_PER_TILE)])

    return pl.kernel(
        body,
        out_type=[
            jax.ShapeDtypeStruct((NP, DW), jnp.float32),
            jax.ShapeDtypeStruct((NP, DW), jnp.float32),
        ],
        mesh=_make_mesh(),
        scratch_types=[
            pltpu.VMEM_SHARED((NP, DW), jnp.float32),
            pltpu.VMEM((4 * DCH, 128), jnp.int32),
            pltpu.VMEM((DCH * 128, DW), jnp.float32),
            pltpu.SemaphoreType.DMA,
            pltpu.SemaphoreType.DMA,
        ],
        compiler_params=_SC_PARAMS,
    )


def _make_propagate(view0, view1):
    def body(tab, srcm, dstm, zrows, out,
             acc, srcb, dstb, rows0, rows1, gsem, ssem, isem):
        c = lax.axis_index("c")
        s = lax.axis_index("s")
        r0 = s * ROWS_PER_TILE
        base = s * ER_PER_TILE
        pltpu.sync_copy(zrows, acc.at[pl.ds(r0, ROWS_PER_TILE)])
        plsc.subcore_barrier()

        def run(tabv):
            def srow(sig, r):
                return srcb.at[sig * BR + r]

            def drow(sig, r):
                return dstb.at[sig * BR + r]

            def fire_gathers(rows_ref, sig, ci):
                for j in range(CH):
                    pltpu.async_copy(tabv.at[srow(sig, CH * ci + j)],
                                     rows_ref.at[pl.ds(j * 128, 128)], gsem)

            def wait_gathers(rows_ref):
                pltpu.make_async_copy(tabv.at[pl.ds(0, CHR)],
                                      rows_ref, gsem).wait()

            def fire_scatters(rows_ref, sig, ci):
                for j in range(CH):
                    pltpu.async_copy(rows_ref.at[pl.ds(j * 128, 128)],
                                     acc.at[drow(sig, CH * ci + j)],
                                     ssem, add=True)

            def wait_scatters(rows_ref):
                pltpu.make_async_copy(rows_ref, acc.at[pl.ds(0, CHR)],
                                      ssem).wait()

            def prefetch_idx(k_next, sig_next):
                rr = base + k_next * BR
                pltpu.async_copy(srcm.at[pl.ds(rr, BR)],
                                 srcb.at[pl.ds(sig_next * BR, BR)], isem)
                pltpu.async_copy(dstm.at[pl.ds(rr, BR)],
                                 dstb.at[pl.ds(sig_next * BR, BR)], isem)

            def wait_idx():
                pltpu.make_async_copy(srcm.at[pl.ds(0, 2 * BR)],
                                      srcb, isem).wait()

            def do_block(k, sig, first):
                for ci in range(BCH):
                    b = ci % 2
                    rows_b = rows0 if b == 0 else rows1
                    rows_nb = rows1 if b == 0 else rows0
                    wait_gathers(rows_b)
                    if ci == 0:
                        if first:
                            prefetch_idx(1, 1)
                        else:
                            wait_scatters(rows_nb)

                            @pl.when(k < NBLK - 1)
                            def _():
                                prefetch_idx(k + 1, 1 - sig)
                    else:
                        wait_scatters(rows_nb)
                    if ci == BCH - 1:
                        if first:
                            wait_idx()
                            fire_gathers(rows_nb, 1, 0)
                        else:
                            @pl.when(k < NBLK - 1)
                            def _():
                                wait_idx()
                                fire_gathers(rows_nb, 1 - sig, 0)
                    else:
                        fire_gathers(rows_nb, sig, ci + 1)
                    fire_scatters(rows_b, sig, ci)

            pltpu.sync_copy(srcm.at[pl.ds(base, BR)], srcb.at[pl.ds(0, BR)])
            pltpu.sync_copy(dstm.at[pl.ds(base, BR)], dstb.at[pl.ds(0, BR)])
            fire_gathers(rows0, 0, 0)
            do_block(0, 0, True)

            def outer(k, carry):
                do_block(k, lax.rem(k, 2), False)
                return carry

            lax.fori_loop(1, NBLK, outer, 0)
            wait_scatters(rows1)

        @pl.when(c == 0)
        def _():
            run(view0(tab))

        @pl.when(c == 1)
        def _():
            run(view1(tab))

        plsc.subcore_barrier()

        @pl.when(c == 0)
        def _():
            pltpu.sync_copy(acc.at[pl.ds(r0, ROWS_PER_TILE)],
                            out.at[0].at[pl.ds(r0, ROWS_PER_TILE)])

        @pl.when(c == 1)
        def _():
            pltpu.sync_copy(acc.at[pl.ds(r0, ROWS_PER_TILE)],
                            out.at[1].at[pl.ds(r0, ROWS_PER_TILE)])

    return pl.kernel(
        body,
        out_type=[jax.ShapeDtypeStruct((NC, NP, FH), jnp.float32)],
        mesh=_make_mesh(),
        scratch_types=[
            pltpu.VMEM_SHARED((NP, FH), jnp.float32),
            pltpu.VMEM((2 * BR, 128), jnp.int32),
            pltpu.VMEM((2 * BR, 128), jnp.int32),
            pltpu.VMEM((CHR, FH), jnp.float32),
            pltpu.VMEM((CHR, FH), jnp.float32),
            pltpu.SemaphoreType.DMA,
            pltpu.SemaphoreType.DMA,
            pltpu.SemaphoreType.DMA,
        ],
        compiler_params=_SC_PARAMS,
    )


_deg = _make_deg()
_prop1 = _make_propagate(
    lambda tab: tab, lambda tab: tab.at[pl.ds(1, 2 * N - 1)])
_prop2 = _make_propagate(
    lambda tab: tab.at[0], lambda tab: tab.at[1])


def _divide(a, deg0, deg1):
    R = NP // 8

    def body(a_ref, d0_ref, d1_ref, h_ref):
        recip = 1.0 / jnp.maximum(d0_ref[..., :1] + d1_ref[..., :1], 1.0)
        h_ref[...] = a_ref[...] * recip

    return pl.pallas_call(
        body,
        grid=(NC, NP // R),
        in_specs=[
            pl.BlockSpec((1, R, FH), lambda c, i: (c, i, 0)),
            pl.BlockSpec((R, DW), lambda c, i: (i, 0)),
            pl.BlockSpec((R, DW), lambda c, i: (i, 0)),
        ],
        out_specs=pl.BlockSpec((1, R, FH), lambda c, i: (c, i, 0)),
        out_shape=jax.ShapeDtypeStruct((NC, NP, FH), jnp.float32),
    )(a, deg0, deg1)


def _scores(xcat, h, b, deg0, deg1, xu, hu, bu, d0u, d1u):
    R = 2000
    OFF = NUM_USERS // R

    def body(x_r, h0_r, h1_r, b0_r, b1_r, d0_r, d1_r,
             xu_r, hu_r, bu_r, d0u_r, d1u_r, out):
        recip = 1.0 / jnp.maximum(d0_r[..., :1] + d1_r[..., :1], 1.0)
        ru = 1.0 / jnp.maximum(d0u_r[..., :1] + d1u_r[..., :1], 1.0)
        xu_b = xu_r[...]
        u0 = (xu_b[:, :FH] + hu_r[0] + bu_r[0] * ru) / 3.0
        u1 = (xu_b[:, FH:] + hu_r[1] + bu_r[1] * ru) / 3.0
        x_b = x_r[...]
        f0 = (x_b[:, :FH] + h0_r[0] + b0_r[0] * recip) / 3.0
        f1 = (x_b[:, FH:] + h1_r[0] + b1_r[0] * recip) / 3.0
        bf = lambda v: v.astype(jnp.bfloat16).astype(jnp.float32)
        t = bf(f0) * bf(u0) + bf(f1) * bf(u1)
        w = FH
        while w > 1:
            w //= 2
            t = t[:, :w] + t[:, w:]
        out[...] = t

    itm = lambda i: (i + OFF, 0)
    itm3a = lambda i: (0, i + OFF, 0)
    itm3b = lambda i: (1, i + OFF, 0)
    rep = lambda i: (0, 0)
    rep3 = lambda i: (0, 0, 0)
    return pl.pallas_call(
        body,
        grid=(NUM_ITEMS // R,),
        in_specs=[
            pl.BlockSpec((R, F), itm),
            pl.BlockSpec((1, R, FH), itm3a),
            pl.BlockSpec((1, R, FH), itm3b),
            pl.BlockSpec((1, R, FH), itm3a),
            pl.BlockSpec((1, R, FH), itm3b),
            pl.BlockSpec((R, DW), itm),
            pl.BlockSpec((R, DW), itm),
            pl.BlockSpec((1, F), rep),
            pl.BlockSpec((NC, 1, FH), rep3),
            pl.BlockSpec((NC, 1, FH), rep3),
            pl.BlockSpec((1, DW), rep),
            pl.BlockSpec((1, DW), rep),
        ],
        out_specs=pl.BlockSpec((R, 1), lambda i: (i, 0)),
        out_shape=jax.ShapeDtypeStruct((NUM_ITEMS, 1), jnp.float32),
    )(xcat, h, h, b, b, deg0, deg1, xu, hu, bu, d0u, d1u)


def _topk(s2d):
    def body(s_ref, vals_ref, idx_ref):
        s = s_ref[...]
        ridx = lax.broadcasted_iota(jnp.int32, s.shape, 0)
        cidx = lax.broadcasted_iota(jnp.int32, s.shape, 1)
        flat = ridx * 128 + cidx
        lane = lax.broadcasted_iota(jnp.int32, (1, 128), 1)
        vals = jnp.zeros((1, 128), jnp.float32)
        idxs = jnp.zeros((1, 128), jnp.int32)
        for k in range(TOPK):
            m = jnp.max(s)
            j = jnp.min(jnp.where(s == m, flat, jnp.int32(2**31 - 1)))
            vals = jnp.where(lane == k, m, vals)
            idxs = jnp.where(lane == k, j, idxs)
            s = jnp.where(flat == j, NEG, s)
        vals_ref[...] = vals
        idx_ref[...] = idxs

    return pl.pallas_call(
        body,
        out_shape=[
            jax.ShapeDtypeStruct((1, 128), jnp.float32),
            jax.ShapeDtypeStruct((1, 128), jnp.int32),
        ],
    )(s2d)


def kernel(user_table, item_table, edge_index, user_id, topk):
    del topk
    xcat = jnp.concatenate([user_table, item_table], axis=0)
    tab1 = xcat.reshape(2 * N, FH)
    src_pad = jnp.zeros((EP - E,), dtype=jnp.int32)
    srcp = jnp.concatenate([edge_index[0], src_pad])
    s1m = (srcp * 2).reshape(ER, 128)
    s2m = srcp.reshape(ER, 128)
    dstm = jnp.concatenate(
        [edge_index[1], jnp.full((EP - E,), NP - 1, dtype=jnp.int32)]
    ).reshape(ER, 128)
    zrows = jnp.zeros((ROWS_PER_TILE, FH), jnp.float32)
    zdeg = jnp.zeros((ROWS_PER_TILE, DW), jnp.float32)
    ones = jnp.ones((DCH * 128, DW), jnp.float32)

    deg0, deg1 = _deg(dstm, zdeg, ones)
    (a,) = _prop1(tab1, s1m, dstm, zrows)
    h = _divide(a, deg0, deg1)
    (b,) = _prop2(h, s2m, dstm, zrows)

    uid = jnp.asarray(user_id, jnp.int32)
    xu = lax.dynamic_slice(xcat, (uid, 0), (1, F))
    hu = lax.dynamic_slice(h, (0, uid, 0), (NC, 1, FH))
    bu = lax.dynamic_slice(b, (0, uid, 0), (NC, 1, FH))
    d0u = lax.dynamic_slice(deg0, (uid, 0), (1, DW))
    d1u = lax.dynamic_slice(deg1, (uid, 0), (1, DW))
    scores = _scores(xcat, h, b, deg0, deg1, xu, hu, bu, d0u, d1u)
    s2d = jnp.pad(scores[:, 0], (0, 40064 - NUM_ITEMS),
                  constant_values=NEG).reshape(313, 128)
    vals, idx = _topk(s2d)
    return vals[0, :TOPK], idx[0, :TOPK]


# speedup vs baseline: 10.0313x; 1.0006x over previous
import jax
import jax.numpy as jnp
from jax import lax
from jax.experimental import pallas as pl
from jax.experimental.pallas import tpu as pltpu
from jax.experimental.pallas import tpu_sc as plsc

NUM_USERS = 10000
NUM_ITEMS = 40000
F = 64
FH = 32
N = NUM_USERS + NUM_ITEMS
E = 800000
NC = 2
NS = 16
NP = 50048
ROWS_PER_TILE = NP // NS
EP = 802816
ER = EP // 128
ER_PER_TILE = ER // NS
CH = 2
CHR = CH * 128
NCHUNK = ER_PER_TILE // CH
BCH = 4
NBLK = NCHUNK // BCH
BR = BCH * CH
DW = 8
ER_PER_CORE = ER // NC
ERD_PER_TILE = ER_PER_CORE // NS
DCH = 4
TOPK = 10
NEG = -1e30

_SC_PARAMS = pltpu.CompilerParams(use_tc_tiling_on_sc=False)


def _make_mesh():
    return plsc.VectorSubcoreMesh(
        core_axis_name="c", subcore_axis_name="s", num_cores=NC
    )


def _make_deg():
    NDBLK = ERD_PER_TILE // DCH

    def body(dstm, zrows, ones_h, out0, out1, degacc, dstb, ones_v, ssem, isem):
        c = lax.axis_index("c")
        s = lax.axis_index("s")
        r0 = s * ROWS_PER_TILE
        base = c * ER_PER_CORE + s * ERD_PER_TILE
        pltpu.sync_copy(zrows, degacc.at[pl.ds(r0, ROWS_PER_TILE)])
        pltpu.sync_copy(ones_h, ones_v)
        plsc.subcore_barrier()

        def prefetch(k_next, sig_next):
            pltpu.async_copy(dstm.at[pl.ds(base + k_next * DCH, DCH)],
                             dstb.at[pl.ds(sig_next * DCH, DCH)], isem)

        def wait_idx(sig):
            pltpu.make_async_copy(dstm.at[pl.ds(base, DCH)],
                                  dstb.at[pl.ds(sig * DCH, DCH)], isem).wait()

        def fire(sig):
            for j in range(DCH):
                pltpu.async_copy(ones_v.at[pl.ds(j * 128, 128)],
                                 degacc.at[dstb.at[sig * DCH + j]],
                                 ssem, add=True)

        def drain():
            pltpu.make_async_copy(ones_v, degacc.at[pl.ds(0, DCH * 128)],
                                  ssem).wait()

        pltpu.sync_copy(dstm.at[pl.ds(base, DCH)], dstb.at[pl.ds(0, DCH)])
        prefetch(1, 1)
        fire(0)

        def outer(k, carry):
            sig = lax.rem(k, 4)
            wait_idx(sig)

            @pl.when(k >= 2)
            def _():
                drain()

            @pl.when(k < NDBLK - 1)
            def _():
                prefetch(k + 1, lax.rem(k + 1, 4))

            fire(sig)
            return carry

        lax.fori_loop(1, NDBLK, outer, 0)
        drain()
        drain()
        plsc.subcore_barrier()

        @pl.when(c == 0)
        def _():
            pltpu.sync_copy(degacc.at[pl.ds(r0, ROWS_PER_TILE)],
                            out0.at[pl.ds(r0, ROWS_PER_TILE)])

        @pl.when(c == 1)
        def _():
            pltpu.sync_copy(degacc.at[pl.ds(r0, ROWS_PER_TILE)],
                            out1.at[pl.ds(r0, ROWS_PER_TILE)])

    return pl.kernel(
        body,
        out_type=[
            jax.ShapeDtypeStruct((NP, DW), jnp.float32),
            jax.ShapeDtypeStruct((NP, DW), jnp.float32),
        ],
        mesh=_make_mesh(),
        scratch_types=[
            pltpu.VMEM_SHARED((NP, DW), jnp.float32),
            pltpu.VMEM((4 * DCH, 128), jnp.int32),
            pltpu.VMEM((DCH * 128, DW), jnp.float32),
            pltpu.SemaphoreType.DMA,
            pltpu.SemaphoreType.DMA,
        ],
        compiler_params=_SC_PARAMS,
    )


def _make_propagate(view0, view1):
    def body(tab, srcm, dstm, zrows, out,
             acc, srcb, dstb, rows0, rows1, gsem, ssem, isem):
        c = lax.axis_index("c")
        s = lax.axis_index("s")
        r0 = s * ROWS_PER_TILE
        base = s * ER_PER_TILE
        pltpu.sync_copy(zrows, acc.at[pl.ds(r0, ROWS_PER_TILE)])
        plsc.subcore_barrier()

        def run(tabv):
            def srow(sig, r):
                return srcb.at[sig * BR + r]

            def drow(sig, r):
                return dstb.at[sig * BR + r]

            def fire_gathers(rows_ref, sig, ci):
                for j in range(CH):
                    pltpu.async_copy(tabv.at[srow(sig, CH * ci + j)],
                                     rows_ref.at[pl.ds(j * 128, 128)], gsem)

            def wait_gathers(rows_ref):
                pltpu.make_async_copy(tabv.at[pl.ds(0, CHR)],
                                      rows_ref, gsem).wait()

            def fire_scatters(rows_ref, sig, ci):
                for j in range(CH):
                    pltpu.async_copy(rows_ref.at[pl.ds(j * 128, 128)],
                                     acc.at[drow(sig, CH * ci + j)],
                                     ssem, add=True)

            def wait_scatters(rows_ref):
                pltpu.make_async_copy(rows_ref, acc.at[pl.ds(0, CHR)],
                                      ssem).wait()

            def prefetch_idx(k_next, sig_next):
                rr = base + k_next * BR
                pltpu.async_copy(srcm.at[pl.ds(rr, BR)],
                                 srcb.at[pl.ds(sig_next * BR, BR)], isem)
                pltpu.async_copy(dstm.at[pl.ds(rr, BR)],
                                 dstb.at[pl.ds(sig_next * BR, BR)], isem)

            def wait_idx():
                pltpu.make_async_copy(srcm.at[pl.ds(0, 2 * BR)],
                                      srcb, isem).wait()

            def do_block(k, sig, first):
                for ci in range(BCH):
                    b = ci % 2
                    rows_b = rows0 if b == 0 else rows1
                    rows_nb = rows1 if b == 0 else rows0
                    wait_gathers(rows_b)
                    if ci == 0:
                        if first:
                            prefetch_idx(1, 1)
                        else:
                            wait_scatters(rows_nb)

                            @pl.when(k < NBLK - 1)
                            def _():
                                prefetch_idx(k + 1, 1 - sig)
                    else:
                        wait_scatters(rows_nb)
                    if ci == BCH - 1:
                        if first:
                            wait_idx()
                            fire_gathers(rows_nb, 1, 0)
                        else:
                            @pl.when(k < NBLK - 1)
                            def _():
                                wait_idx()
                                fire_gathers(rows_nb, 1 - sig, 0)
                    else:
                        fire_gathers(rows_nb, sig, ci + 1)
                    fire_scatters(rows_b, sig, ci)

            pltpu.sync_copy(srcm.at[pl.ds(base, BR)], srcb.at[pl.ds(0, BR)])
            pltpu.sync_copy(dstm.at[pl.ds(base, BR)], dstb.at[pl.ds(0, BR)])
            fire_gathers(rows0, 0, 0)
            do_block(0, 0, True)

            def outer(k, carry):
                do_block(k, lax.rem(k, 2), False)
                return carry

            lax.fori_loop(1, NBLK, outer, 0)
            wait_scatters(rows1)

        @pl.when(c == 0)
        def _():
            run(view0(tab))

        @pl.when(c == 1)
        def _():
            run(view1(tab))

        plsc.subcore_barrier()

        @pl.when(c == 0)
        def _():
            pltpu.sync_copy(acc.at[pl.ds(r0, ROWS_PER_TILE)],
                            out.at[0].at[pl.ds(r0, ROWS_PER_TILE)])

        @pl.when(c == 1)
        def _():
            pltpu.sync_copy(acc.at[pl.ds(r0, ROWS_PER_TILE)],
                            out.at[1].at[pl.ds(r0, ROWS_PER_TILE)])

    return pl.kernel(
        body,
        out_type=[jax.ShapeDtypeStruct((NC, NP, FH), jnp.float32)],
        mesh=_make_mesh(),
        scratch_types=[
            pltpu.VMEM_SHARED((NP, FH), jnp.float32),
            pltpu.VMEM((2 * BR, 128), jnp.int32),
            pltpu.VMEM((2 * BR, 128), jnp.int32),
            pltpu.VMEM((CHR, FH), jnp.float32),
            pltpu.VMEM((CHR, FH), jnp.float32),
            pltpu.SemaphoreType.DMA,
            pltpu.SemaphoreType.DMA,
            pltpu.SemaphoreType.DMA,
        ],
        compiler_params=_SC_PARAMS,
    )


_deg = _make_deg()
_prop1 = _make_propagate(
    lambda tab: tab, lambda tab: tab.at[pl.ds(1, 2 * N - 1)])
_prop2 = _make_propagate(
    lambda tab: tab.at[0], lambda tab: tab.at[1])


def _divide(a, deg0, deg1):
    R = NP // 8

    def body(a_ref, d0_ref, d1_ref, h_ref):
        recip = 1.0 / jnp.maximum(d0_ref[..., :1] + d1_ref[..., :1], 1.0)
        h_ref[...] = a_ref[...] * recip

    return pl.pallas_call(
        body,
        grid=(NC, NP // R),
        in_specs=[
            pl.BlockSpec((1, R, FH), lambda c, i: (c, i, 0)),
            pl.BlockSpec((R, DW), lambda c, i: (i, 0)),
            pl.BlockSpec((R, DW), lambda c, i: (i, 0)),
        ],
        out_specs=pl.BlockSpec((1, R, FH), lambda c, i: (c, i, 0)),
        out_shape=jax.ShapeDtypeStruct((NC, NP, FH), jnp.float32),
    )(a, deg0, deg1)


def _scores(xcat, h, b, deg0, deg1, xu, hu, bu, d0u, d1u):
    R = 2000
    OFF = NUM_USERS // R

    def body(x_r, h0_r, h1_r, b0_r, b1_r, d0_r, d1_r,
             xu_r, hu_r, bu_r, d0u_r, d1u_r, out):
        recip = 1.0 / jnp.maximum(d0_r[..., :1] + d1_r[..., :1], 1.0)
        ru = 1.0 / jnp.maximum(d0u_r[..., :1] + d1u_r[..., :1], 1.0)
        xu_b = xu_r[...]
        u0 = (xu_b[:, :FH] + hu_r[0] + bu_r[0] * ru) / 3.0
        u1 = (xu_b[:, FH:] + hu_r[1] + bu_r[1] * ru) / 3.0
        x_b = x_r[...]
        f0 = (x_b[:, :FH] + h0_r[0] + b0_r[0] * recip) / 3.0
        f1 = (x_b[:, FH:] + h1_r[0] + b1_r[0] * recip) / 3.0
        bf = lambda v: v.astype(jnp.bfloat16).astype(jnp.float32)
        t = bf(f0) * bf(u0) + bf(f1) * bf(u1)
        w = FH
        while w > 1:
            w //= 2
            t = t[:, :w] + t[:, w:]
        out[...] = t

    itm = lambda i: (i + OFF, 0)
    itm3a = lambda i: (0, i + OFF, 0)
    itm3b = lambda i: (1, i + OFF, 0)
    rep = lambda i: (0, 0)
    rep3 = lambda i: (0, 0, 0)
    return pl.pallas_call(
        body,
        grid=(NUM_ITEMS // R,),
        in_specs=[
            pl.BlockSpec((R, F), itm),
            pl.BlockSpec((1, R, FH), itm3a),
            pl.BlockSpec((1, R, FH), itm3b),
            pl.BlockSpec((1, R, FH), itm3a),
            pl.BlockSpec((1, R, FH), itm3b),
            pl.BlockSpec((R, DW), itm),
            pl.BlockSpec((R, DW), itm),
            pl.BlockSpec((1, F), rep),
            pl.BlockSpec((NC, 1, FH), rep3),
            pl.BlockSpec((NC, 1, FH), rep3),
            pl.BlockSpec((1, DW), rep),
            pl.BlockSpec((1, DW), rep),
        ],
        out_specs=pl.BlockSpec((R, 1), lambda i: (i, 0)),
        out_shape=jax.ShapeDtypeStruct((NUM_ITEMS, 1), jnp.float32),
    )(xcat, h, h, b, b, deg0, deg1, xu, hu, bu, d0u, d1u)


def _topk(s2d):
    def body(s_ref, vals_ref, idx_ref):
        s = s_ref[...]
        ridx = lax.broadcasted_iota(jnp.int32, s.shape, 0)
        cidx = lax.broadcasted_iota(jnp.int32, s.shape, 1)
        flat = ridx * 128 + cidx
        lane = lax.broadcasted_iota(jnp.int32, (1, 128), 1)
        vals = jnp.zeros((1, 128), jnp.float32)
        idxs = jnp.zeros((1, 128), jnp.int32)
        for k in range(TOPK):
            m = jnp.max(s)
            j = jnp.min(jnp.where(s == m, flat, jnp.int32(2**31 - 1)))
            vals = jnp.where(lane == k, m, vals)
            idxs = jnp.where(lane == k, j, idxs)
            s = jnp.where(flat == j, NEG, s)
        vals_ref[...] = vals
        idx_ref[...] = idxs

    return pl.pallas_call(
        body,
        out_shape=[
            jax.ShapeDtypeStruct((1, 128), jnp.float32),
            jax.ShapeDtypeStruct((1, 128), jnp.int32),
        ],
    )(s2d)


def kernel(user_table, item_table, edge_index, user_id, topk):
    del topk
    xcat = jnp.concatenate([user_table, item_table], axis=0)
    tab1 = xcat.reshape(2 * N, FH)
    src_pad = jnp.zeros((EP - E,), dtype=jnp.int32)
    srcp = jnp.concatenate([edge_index[0], src_pad])
    s1m = (srcp * 2).reshape(ER, 128)
    s2m = srcp.reshape(ER, 128)
    dstm = jnp.concatenate(
        [edge_index[1], jnp.full((EP - E,), NP - 1, dtype=jnp.int32)]
    ).reshape(ER, 128)
    zrows = jnp.zeros((ROWS_PER_TILE, FH), jnp.float32)
    zdeg = jnp.zeros((ROWS_PER_TILE, DW), jnp.float32)
    ones = jnp.ones((DCH * 128, DW), jnp.float32)

    deg0, deg1 = _deg(dstm, zdeg, ones)
    (a,) = _prop1(tab1, s1m, dstm, zrows)
    h = _divide(a, deg0, deg1)
    (b,) = _prop2(h, s2m, dstm, zrows)

    uid = jnp.asarray(user_id, jnp.int32)
    xu = lax.dynamic_slice(xcat, (uid, 0), (1, F))
    hu = lax.dynamic_slice(h, (0, uid, 0), (NC, 1, FH))
    bu = lax.dynamic_slice(b, (0, uid, 0), (NC, 1, FH))
    d0u = lax.dynamic_slice(deg0, (uid, 0), (1, DW))
    d1u = lax.dynamic_slice(deg1, (uid, 0), (1, DW))
    scores = _scores(xcat, h, b, deg0, deg1, xu, hu, bu, d0u, d1u)
    s2d = jnp.pad(scores[:, 0], (0, 40064 - NUM_ITEMS),
                  constant_values=NEG).reshape(313, 128)
    vals, idx = _topk(s2d)
    return vals[0, :TOPK], idx[0, :TOPK]
